# baseline (speedup 1.0000x reference)
import functools

import jax
import jax.numpy as jnp
import numpy as np
from jax import lax
from jax.experimental import pallas as pl
from jax.experimental.pallas import tpu as pltpu
from jax.experimental.pallas import tpu_sc as plsc

H, W, K, N_ITERS = 224, 224, 64, 5
NC = 2
NS = 16
ROWS_PER = H // (NC * NS)
VPR = W // 16
ACC = 2 * K * 16

_MAGIC = np.int32(0x5F3759DF)


def _rsqrt(x):
    i = plsc.bitcast(x, jnp.int32)
    y = plsc.bitcast(_MAGIC - (i >> 1), jnp.float32)
    for _ in range(2):
        y = y * (jnp.float32(1.5) - jnp.float32(0.5) * x * y * y)
    return y


def _bcast(v, j):
    return v.at[jnp.full((16,), j, jnp.int32)].get(mode="promise_in_bounds")


def _body(cl_hbm, hm_hbm, out_hbm, part_hbm, flag_hbm,
          hm_v, cl_v, acc_v, big_v, ncl_v, fl_v, both_v, stage_s, ncl_s):
    cid = lax.axis_index("c")
    sid = lax.axis_index("s")
    tid = cid * NS + sid
    lane = lax.iota(jnp.int32, 16)

    @pl.when(sid == 0)
    def _():
        fl_v[...] = jnp.zeros((16,), jnp.int32)
        pltpu.sync_copy(fl_v, flag_hbm.at[cid])

    pltpu.sync_copy(hm_hbm.at[pl.ds(tid * (ROWS_PER * W), ROWS_PER * W)],
                    hm_v)

    pltpu.sync_copy(cl_hbm, ncl_v)
    for g in range(4):
        idx = (lane + 16 * g) * 2
        cl_v[pl.ds(16 * g, 16)] = plsc.load_gather(ncl_v, [idx])
        cl_v[pl.ds(K + 16 * g, 16)] = plsc.load_gather(ncl_v, [idx + 1])

    def iteration(it, carry):
        def zero_body(j, c):
            acc_v[pl.ds(j * 16, 16)] = jnp.zeros((16,), jnp.float32)
            return c
        lax.fori_loop(0, ACC // 16, zero_body, 0)

        crow = [cl_v[pl.ds(16 * g, 16)] for g in range(4)]
        ccol = [cl_v[pl.ds(K + 16 * g, 16)] for g in range(4)]

        NB = 14

        def row_body(r, c):
            rf = (tid * ROWS_PER + r).astype(jnp.float32)
            rvec = jnp.full((16,), rf, jnp.float32)
            a2 = [(rvec - crow[g]) * (rvec - crow[g]) for g in range(4)]

            for blk in range(1):
                ccs = [((blk * NB + i) * 16 + lane).astype(jnp.float32)
                       for i in range(NB)]
                best = [jnp.full((16,), jnp.inf, jnp.float32)
                        for _ in range(NB)]
                bk = [jnp.zeros((16,), jnp.int32) for _ in range(NB)]
                for g in range(4):
                    def jbody(j, carry2, g=g):
                        b = list(carry2[:NB])
                        kk = list(carry2[NB:])
                        a2b = _bcast(a2[g], j)
                        ccb = _bcast(ccol[g], j)
                        kv = jnp.full((16,), j + 16 * g, jnp.int32)
                        for i in range(NB):
                            dc = ccs[i] - ccb
                            e2 = jnp.maximum(jnp.float32(1.0),
                                             dc * dc + a2b)
                            m = e2 < b[i]
                            b[i] = jnp.where(m, e2, b[i])
                            kk[i] = jnp.where(m, kv, kk[i])
                        return (*b, *kk)
                    out = lax.fori_loop(0, 16, jbody, (*best, *bk),
                                        unroll=2)
                    best = list(out[:NB])
                    bk = list(out[NB:])
                for i in range(NB):
                    px = (blk * NB + i) * 16
                    hmv = hm_v[pl.ds(r * W + px, 16)]
                    w = hmv * _rsqrt(best[i])
                    idx_r = bk[i] * 16 + lane
                    plsc.addupdate_scatter(acc_v, [idx_r], rvec * w)
                    plsc.addupdate_scatter(acc_v, [idx_r + K * 16],
                                           ccs[i] * w)
            return c
        lax.fori_loop(0, ROWS_PER, row_body, 0)

        pltpu.sync_copy(acc_v, stage_s.at[pl.ds(sid * ACC, ACC)])
        plsc.subcore_barrier()

        @pl.when(sid < 8)
        def _():
            for s in range(NS):
                pltpu.sync_copy(
                    stage_s.at[pl.ds(s * ACC + sid * 256, 256)],
                    big_v.at[pl.ds(s * 256, 256)])

            def red_body(t, c):
                v = big_v[pl.ds(t * 16, 16)]
                for s in range(1, NS):
                    v = v + big_v[pl.ds(s * 256 + t * 16, 16)]
                for sh in (8, 4, 2, 1):
                    v = v + v.at[lane ^ sh].get(mode="promise_in_bounds")
                plsc.store_scatter(
                    ncl_v, [jnp.full((16,), 16, jnp.int32) * sid + t], v,
                    mask=lane == 0)
                return c
            lax.fori_loop(0, 16, red_body, 0)
            pltpu.sync_copy(ncl_v.at[pl.ds(sid * 16, 16)],
                            ncl_s.at[pl.ds(sid * 16, 16)])

        plsc.subcore_barrier()

        @pl.when(sid == 0)
        def _():
            pltpu.sync_copy(ncl_s, part_hbm.at[cid])
            fl_v[...] = jnp.full((16,), it + 1, jnp.int32)
            pltpu.sync_copy(fl_v, flag_hbm.at[cid])

            def cond(seen):
                return seen < it + 1

            def poll(seen):
                pltpu.sync_copy(flag_hbm.at[1 - cid], fl_v)
                v = fl_v[...]
                return v[0]
            lax.while_loop(cond, poll, jnp.int32(-1))
            pltpu.sync_copy(part_hbm, both_v)
            for g in range(8):
                s0 = both_v[0, pl.ds(16 * g, 16)]
                s1 = both_v[1, pl.ds(16 * g, 16)]
                ncl_v[pl.ds(16 * g, 16)] = s0 + s1
            pltpu.sync_copy(ncl_v, ncl_s)

        plsc.subcore_barrier()
        pltpu.sync_copy(ncl_s, cl_v)
        return carry
    lax.fori_loop(0, N_ITERS, iteration, 0)

    @pl.when(tid == 0)
    def _():
        for g in range(4):
            idx = (lane + 16 * g) * 2
            plsc.store_scatter(ncl_v, [idx], cl_v[pl.ds(16 * g, 16)])
            plsc.store_scatter(ncl_v, [idx + 1], cl_v[pl.ds(K + 16 * g, 16)])
        pltpu.sync_copy(ncl_v, out_hbm)


@jax.jit
def _run(cl_flat, hm_flat):
    mesh = plsc.VectorSubcoreMesh(core_axis_name="c", subcore_axis_name="s")
    fn = pl.kernel(
        _body,
        out_type=(jax.ShapeDtypeStruct((2 * K,), jnp.float32),
                  jax.ShapeDtypeStruct((NC, 2 * K), jnp.float32),
                  jax.ShapeDtypeStruct((NC, 16), jnp.int32)),
        mesh=mesh,
        compiler_params=pltpu.CompilerParams(needs_layout_passes=False),
        scratch_types=[
            pltpu.VMEM((ROWS_PER * W,), jnp.float32),
            pltpu.VMEM((2 * K,), jnp.float32),
            pltpu.VMEM((ACC,), jnp.float32),
            pltpu.VMEM((NS * 256,), jnp.float32),
            pltpu.VMEM((2 * K,), jnp.float32),
            pltpu.VMEM((16,), jnp.int32),
            pltpu.VMEM((NC, 2 * K), jnp.float32),
            pltpu.VMEM_SHARED((NS * ACC,), jnp.float32),
            pltpu.VMEM_SHARED((2 * K,), jnp.float32),
        ],
    )
    out, _, _ = fn(cl_flat, hm_flat)
    return out


def kernel(clusters, heatmap):
    hm = heatmap
    if hm.ndim == 3:
        hm = hm[0]
    out = _run(clusters.reshape(-1).astype(jnp.float32),
               hm.reshape(-1).astype(jnp.float32))
    return out.reshape(K, 2)


# speedup vs baseline: 1.0449x; 1.0449x over previous
import jax
import jax.numpy as jnp
import numpy as np
from jax import lax
from jax.experimental import pallas as pl
from jax.experimental.pallas import tpu as pltpu
from jax.experimental.pallas import tpu_sc as plsc

H, W, K, N_ITERS = 224, 224, 64, 5
NC = 2
NS = 16
ROWS_PER = H // (NC * NS)
ACC = 2 * K * 16

_MAGIC = np.int32(0x5F3759DF)


def _rsqrt(x):
    i = plsc.bitcast(x, jnp.int32)
    y = plsc.bitcast(_MAGIC - (i >> 1), jnp.float32)
    for _ in range(2):
        y = y * (jnp.float32(1.5) - jnp.float32(0.5) * x * y * y)
    return y


def _bcast(v, j):
    return v.at[jnp.full((16,), j, jnp.int32)].get(mode="promise_in_bounds")


def _body(cl_hbm, hm_hbm, out_hbm, part_hbm, flag_hbm,
          hm_v, cl_v, acc_v, big_v, ncl_v, fl_v, both_v, stage_s, ncl_s):
    cid = lax.axis_index("c")
    sid = lax.axis_index("s")
    tid = cid * NS + sid
    lane = lax.iota(jnp.int32, 16)

    @pl.when(sid == 0)
    def _():
        fl_v[...] = jnp.zeros((16,), jnp.int32)
        pltpu.sync_copy(fl_v, flag_hbm.at[cid])

    pltpu.sync_copy(hm_hbm.at[pl.ds(tid * (ROWS_PER * W), ROWS_PER * W)],
                    hm_v)

    pltpu.sync_copy(cl_hbm, ncl_v)
    for g in range(4):
        idx = (lane + 16 * g) * 2
        cl_v[pl.ds(16 * g, 16)] = plsc.load_gather(ncl_v, [idx])
        cl_v[pl.ds(K + 16 * g, 16)] = plsc.load_gather(ncl_v, [idx + 1])

    def iteration(it, carry):
        def zero_body(j, c):
            acc_v[pl.ds(j * 16, 16)] = jnp.zeros((16,), jnp.float32)
            return c
        lax.fori_loop(0, ACC // 16, zero_body, 0)

        crow = [cl_v[pl.ds(16 * g, 16)] for g in range(4)]
        ccol = [cl_v[pl.ds(K + 16 * g, 16)] for g in range(4)]

        NB = 7

        def row_body(r, c):
            rf = (tid * ROWS_PER + r).astype(jnp.float32)
            rvec = jnp.full((16,), rf, jnp.float32)
            a2 = [(rvec - crow[g]) * (rvec - crow[g]) for g in range(4)]

            for blk in range(2):
                ccs = [((blk * NB + i) * 16 + lane).astype(jnp.float32)
                       for i in range(NB)]
                best = [jnp.full((16,), jnp.inf, jnp.float32)
                        for _ in range(NB)]
                bk = [jnp.zeros((16,), jnp.int32) for _ in range(NB)]
                for g in range(4):
                    def jbody(j, carry2, g=g):
                        b = list(carry2[:NB])
                        kk = list(carry2[NB:])
                        a2b = _bcast(a2[g], j)
                        ccb = _bcast(ccol[g], j)
                        kv = jnp.full((16,), j + 16 * g, jnp.int32)
                        for i in range(NB):
                            dc = ccs[i] - ccb
                            e2 = jnp.maximum(jnp.float32(1.0),
                                             dc * dc + a2b)
                            m = e2 < b[i]
                            b[i] = jnp.where(m, e2, b[i])
                            kk[i] = jnp.where(m, kv, kk[i])
                        return (*b, *kk)
                    out = lax.fori_loop(0, 16, jbody, (*best, *bk),
                                        unroll=2)
                    best = list(out[:NB])
                    bk = list(out[NB:])
                for i in range(NB):
                    px = (blk * NB + i) * 16
                    hmv = hm_v[pl.ds(r * W + px, 16)]
                    w = hmv * _rsqrt(best[i])
                    idx_r = bk[i] * 16 + lane
                    plsc.addupdate_scatter(acc_v, [idx_r], rvec * w)
                    plsc.addupdate_scatter(acc_v, [idx_r + K * 16],
                                           ccs[i] * w)
            return c
        lax.fori_loop(0, ROWS_PER, row_body, 0)

        pltpu.sync_copy(acc_v, stage_s.at[pl.ds(sid * ACC, ACC)])
        plsc.subcore_barrier()

        @pl.when(sid < 8)
        def _():
            for s in range(NS):
                pltpu.sync_copy(
                    stage_s.at[pl.ds(s * ACC + sid * 256, 256)],
                    big_v.at[pl.ds(s * 256, 256)])

            def red_body(t, c):
                v = big_v[pl.ds(t * 16, 16)]
                for s in range(1, NS):
                    v = v + big_v[pl.ds(s * 256 + t * 16, 16)]
                for sh in (8, 4, 2, 1):
                    v = v + v.at[lane ^ sh].get(mode="promise_in_bounds")
                plsc.store_scatter(
                    ncl_v, [jnp.full((16,), 16, jnp.int32) * sid + t], v,
                    mask=lane == 0)
                return c
            lax.fori_loop(0, 16, red_body, 0)
            pltpu.sync_copy(ncl_v.at[pl.ds(sid * 16, 16)],
                            ncl_s.at[pl.ds(sid * 16, 16)])

        plsc.subcore_barrier()

        @pl.when(sid == 0)
        def _():
            pltpu.sync_copy(ncl_s, part_hbm.at[cid])
            fl_v[...] = jnp.full((16,), it + 1, jnp.int32)
            pltpu.sync_copy(fl_v, flag_hbm.at[cid])

            def cond(seen):
                return seen < it + 1

            def poll(seen):
                pltpu.sync_copy(flag_hbm.at[1 - cid], fl_v)
                v = fl_v[...]
                return v[0]
            lax.while_loop(cond, poll, jnp.int32(-1))
            pltpu.sync_copy(part_hbm, both_v)
            for g in range(8):
                s0 = both_v[0, pl.ds(16 * g, 16)]
                s1 = both_v[1, pl.ds(16 * g, 16)]
                ncl_v[pl.ds(16 * g, 16)] = s0 + s1
            pltpu.sync_copy(ncl_v, ncl_s)

        plsc.subcore_barrier()
        pltpu.sync_copy(ncl_s, cl_v)
        return carry
    lax.fori_loop(0, N_ITERS, iteration, 0)

    @pl.when(tid == 0)
    def _():
        for g in range(4):
            idx = (lane + 16 * g) * 2
            plsc.store_scatter(ncl_v, [idx], cl_v[pl.ds(16 * g, 16)])
            plsc.store_scatter(ncl_v, [idx + 1], cl_v[pl.ds(K + 16 * g, 16)])
        pltpu.sync_copy(ncl_v, out_hbm)


@jax.jit
def _run(cl_flat, hm_flat):
    mesh = plsc.VectorSubcoreMesh(core_axis_name="c", subcore_axis_name="s")
    fn = pl.kernel(
        _body,
        out_type=(jax.ShapeDtypeStruct((2 * K,), jnp.float32),
                  jax.ShapeDtypeStruct((NC, 2 * K), jnp.float32),
                  jax.ShapeDtypeStruct((NC, 16), jnp.int32)),
        mesh=mesh,
        compiler_params=pltpu.CompilerParams(needs_layout_passes=False),
        scratch_types=[
            pltpu.VMEM((ROWS_PER * W,), jnp.float32),
            pltpu.VMEM((2 * K,), jnp.float32),
            pltpu.VMEM((ACC,), jnp.float32),
            pltpu.VMEM((NS * 256,), jnp.float32),
            pltpu.VMEM((2 * K,), jnp.float32),
            pltpu.VMEM((16,), jnp.int32),
            pltpu.VMEM((NC, 2 * K), jnp.float32),
            pltpu.VMEM_SHARED((NS * ACC,), jnp.float32),
            pltpu.VMEM_SHARED((2 * K,), jnp.float32),
        ],
    )
    out, _, _ = fn(cl_flat, hm_flat)
    return out


def kernel(clusters, heatmap):
    hm = heatmap
    if hm.ndim == 3:
        hm = hm[0]
    out = _run(clusters.reshape(-1).astype(jnp.float32),
               hm.reshape(-1).astype(jnp.float32))
    return out.reshape(K, 2)


# speedup vs baseline: 1.1310x; 1.0825x over previous
import jax
import jax.numpy as jnp
import numpy as np
from jax import lax
from jax.experimental import pallas as pl
from jax.experimental.pallas import tpu as pltpu
from jax.experimental.pallas import tpu_sc as plsc

H, W, K, N_ITERS = 224, 224, 64, 5
NC = 2
NS = 16
ROWS_PER = H // (NC * NS)
ACC = 2 * K * 16

_MAGIC = np.int32(0x5F3759DF)


def _rsqrt(x):
    i = plsc.bitcast(x, jnp.int32)
    y = plsc.bitcast(_MAGIC - (i >> 1), jnp.float32)
    for _ in range(2):
        y = y * (jnp.float32(1.5) - jnp.float32(0.5) * x * y * y)
    return y


def _bcast(v, j):
    return v.at[jnp.full((16,), j, jnp.int32)].get(mode="promise_in_bounds")


def _body(cl_hbm, hm_hbm, out_hbm, part_hbm, flag_hbm,
          hm_v, cl_v, acc_v, big_v, ncl_v, fl_v, both_v, stage_s, ncl_s, sem):
    cid = lax.axis_index("c")
    sid = lax.axis_index("s")
    tid = cid * NS + sid
    lane = lax.iota(jnp.int32, 16)

    @pl.when(sid == 0)
    def _():
        fl_v[...] = jnp.zeros((16,), jnp.int32)
        pltpu.sync_copy(fl_v, flag_hbm.at[cid])

    pltpu.sync_copy(hm_hbm.at[pl.ds(tid * (ROWS_PER * W), ROWS_PER * W)],
                    hm_v)

    pltpu.sync_copy(cl_hbm, ncl_v)
    for g in range(4):
        idx = (lane + 16 * g) * 2
        cl_v[pl.ds(16 * g, 16)] = plsc.load_gather(ncl_v, [idx])
        cl_v[pl.ds(K + 16 * g, 16)] = plsc.load_gather(ncl_v, [idx + 1])

    def iteration(it, carry):
        def zero_body(j, c):
            acc_v[pl.ds(j * 16, 16)] = jnp.zeros((16,), jnp.float32)
            return c
        lax.fori_loop(0, ACC // 16, zero_body, 0)

        crow = [cl_v[pl.ds(16 * g, 16)] for g in range(4)]
        ccol = [cl_v[pl.ds(K + 16 * g, 16)] for g in range(4)]

        NB = 7

        def row_body(r, c):
            rf = (tid * ROWS_PER + r).astype(jnp.float32)
            rvec = jnp.full((16,), rf, jnp.float32)
            a2 = [(rvec - crow[g]) * (rvec - crow[g]) for g in range(4)]

            for blk in range(2):
                ccs = [((blk * NB + i) * 16 + lane).astype(jnp.float32)
                       for i in range(NB)]
                best = [jnp.full((16,), jnp.inf, jnp.float32)
                        for _ in range(NB)]
                bk = [jnp.zeros((16,), jnp.int32) for _ in range(NB)]
                for g in range(4):
                    def jbody(j, carry2, g=g):
                        b = list(carry2[:NB])
                        kk = list(carry2[NB:])
                        a2b = _bcast(a2[g], j)
                        ccb = _bcast(ccol[g], j)
                        kv = jnp.full((16,), j + 16 * g, jnp.int32)
                        for i in range(NB):
                            dc = ccs[i] - ccb
                            e2 = jnp.maximum(jnp.float32(1.0),
                                             dc * dc + a2b)
                            m = e2 < b[i]
                            b[i] = jnp.where(m, e2, b[i])
                            kk[i] = jnp.where(m, kv, kk[i])
                        return (*b, *kk)
                    out = lax.fori_loop(0, 16, jbody, (*best, *bk),
                                        unroll=2)
                    best = list(out[:NB])
                    bk = list(out[NB:])
                for i in range(NB):
                    px = (blk * NB + i) * 16
                    hmv = hm_v[pl.ds(r * W + px, 16)]
                    w = hmv * _rsqrt(best[i])
                    idx_r = bk[i] * 16 + lane
                    plsc.addupdate_scatter(acc_v, [idx_r], rvec * w)
                    plsc.addupdate_scatter(acc_v, [idx_r + K * 16],
                                           ccs[i] * w)
            return c
        lax.fori_loop(0, ROWS_PER, row_body, 0)

        pltpu.sync_copy(acc_v, stage_s.at[pl.ds(sid * ACC, ACC)])
        plsc.subcore_barrier()

        @pl.when(sid < 8)
        def _():
            descs = []
            for s in range(NS):
                d = pltpu.make_async_copy(
                    stage_s.at[pl.ds(s * ACC + sid * 256, 256)],
                    big_v.at[pl.ds(s * 256, 256)], sem)
                d.start()
                descs.append(d)
            for d in descs:
                d.wait()

            def red_body(t, c):
                v = big_v[pl.ds(t * 16, 16)]
                for s in range(1, NS):
                    v = v + big_v[pl.ds(s * 256 + t * 16, 16)]
                for sh in (8, 4, 2, 1):
                    v = v + v.at[lane ^ sh].get(mode="promise_in_bounds")
                plsc.store_scatter(
                    ncl_v, [jnp.full((16,), 16, jnp.int32) * sid + t], v,
                    mask=lane == 0)
                return c
            lax.fori_loop(0, 16, red_body, 0)
            pltpu.sync_copy(ncl_v.at[pl.ds(sid * 16, 16)],
                            ncl_s.at[pl.ds(sid * 16, 16)])

        plsc.subcore_barrier()

        @pl.when(sid == 0)
        def _():
            pltpu.sync_copy(ncl_s, part_hbm.at[cid])
            fl_v[...] = jnp.full((16,), it + 1, jnp.int32)
            pltpu.sync_copy(fl_v, flag_hbm.at[cid])

            def cond(seen):
                return seen < it + 1

            def poll(seen):
                pltpu.sync_copy(flag_hbm.at[1 - cid], fl_v)
                v = fl_v[...]
                return v[0]
            lax.while_loop(cond, poll, jnp.int32(-1))
            pltpu.sync_copy(part_hbm, both_v)
            for g in range(8):
                s0 = both_v[0, pl.ds(16 * g, 16)]
                s1 = both_v[1, pl.ds(16 * g, 16)]
                ncl_v[pl.ds(16 * g, 16)] = s0 + s1
            pltpu.sync_copy(ncl_v, ncl_s)

        plsc.subcore_barrier()
        pltpu.sync_copy(ncl_s, cl_v)
        return carry
    lax.fori_loop(0, N_ITERS, iteration, 0)

    @pl.when(tid == 0)
    def _():
        for g in range(4):
            idx = (lane + 16 * g) * 2
            plsc.store_scatter(ncl_v, [idx], cl_v[pl.ds(16 * g, 16)])
            plsc.store_scatter(ncl_v, [idx + 1], cl_v[pl.ds(K + 16 * g, 16)])
        pltpu.sync_copy(ncl_v, out_hbm)


@jax.jit
def _run(cl_flat, hm_flat):
    mesh = plsc.VectorSubcoreMesh(core_axis_name="c", subcore_axis_name="s")
    fn = pl.kernel(
        _body,
        out_type=(jax.ShapeDtypeStruct((2 * K,), jnp.float32),
                  jax.ShapeDtypeStruct((NC, 2 * K), jnp.float32),
                  jax.ShapeDtypeStruct((NC, 16), jnp.int32)),
        mesh=mesh,
        compiler_params=pltpu.CompilerParams(needs_layout_passes=False),
        scratch_types=[
            pltpu.VMEM((ROWS_PER * W,), jnp.float32),
            pltpu.VMEM((2 * K,), jnp.float32),
            pltpu.VMEM((ACC,), jnp.float32),
            pltpu.VMEM((NS * 256,), jnp.float32),
            pltpu.VMEM((2 * K,), jnp.float32),
            pltpu.VMEM((16,), jnp.int32),
            pltpu.VMEM((NC, 2 * K), jnp.float32),
            pltpu.VMEM_SHARED((NS * ACC,), jnp.float32),
            pltpu.VMEM_SHARED((2 * K,), jnp.float32),
            pltpu.SemaphoreType.DMA,
        ],
    )
    out, _, _ = fn(cl_flat, hm_flat)
    return out


def kernel(clusters, heatmap):
    hm = heatmap
    if hm.ndim == 3:
        hm = hm[0]
    out = _run(clusters.reshape(-1).astype(jnp.float32),
               hm.reshape(-1).astype(jnp.float32))
    return out.reshape(K, 2)
